# Initial kernel scaffold; baseline (speedup 1.0000x reference)
import jax
import jax.numpy as jnp
from jax.experimental import pallas as pl


def kernel(slot_features, W, b, basis_vectors):
    raise NotImplementedError("write your pallas kernel here")


# speedup vs baseline: 1.2243x; 1.2243x over previous
import functools

import jax
import jax.numpy as jnp
from jax import lax
from jax.experimental import pallas as pl
from jax.experimental.pallas import tpu as pltpu
from jax.experimental.pallas import tpu_sc as plsc

_NUM_CODES = 1024
_BASIS_DIM = 2700
_BETA = 0.25
_BLK = 256


def _vq_tc_body(slots_ref, w_ref, b_ref, basis_ref,
                idx_ref, loss_ref, ent_ref,
                acc_ref, cn_ref, msum_ref):
    i = pl.program_id(0)
    nblk = pl.num_programs(0)

    @pl.when(i == 0)
    def _init():
        bsq = basis_ref[...] * basis_ref[...]
        cn_ref[...] = jnp.sum(bsq, axis=1)[None, :]
        acc_ref[...] = jnp.zeros_like(acc_ref)
        msum_ref[0] = 0.0

    z = jnp.dot(slots_ref[...], w_ref[...],
                preferred_element_type=jnp.float32) + b_ref[...]
    g = lax.dot_general(z, basis_ref[...], (((1,), (1,)), ((), ())),
                        preferred_element_type=jnp.float32)
    zn = jnp.sum(z * z, axis=1, keepdims=True)
    dist = zn - 2.0 * g + cn_ref[...]
    m = jnp.min(dist, axis=1, keepdims=True)
    idx_ref[0, 0, :] = jnp.argmin(dist, axis=1).astype(jnp.int32)
    p = jnp.exp(m - dist)
    p = p / jnp.sum(p, axis=1, keepdims=True)
    acc_ref[...] += jnp.sum(p, axis=0, keepdims=True)
    msum_ref[0] += jnp.sum(m)

    @pl.when(i == nblk - 1)
    def _fin():
        nrows = nblk * _BLK
        avg = acc_ref[...] / nrows
        ent_ref[0, 0] = -jnp.sum(avg * jnp.log(avg + 1e-8))
        loss_ref[0, 0] = (_BETA / (nrows * _BASIS_DIM)) * msum_ref[0]


def _vq_tc(slots2d, w, b2, basis):
    nrows, d = slots2d.shape
    nblk = nrows // _BLK
    return pl.pallas_call(
        _vq_tc_body,
        grid=(nblk,),
        in_specs=[
            pl.BlockSpec((_BLK, d), lambda i: (i, 0)),
            pl.BlockSpec(w.shape, lambda i: (0, 0)),
            pl.BlockSpec(b2.shape, lambda i: (0, 0)),
            pl.BlockSpec(basis.shape, lambda i: (0, 0)),
        ],
        out_specs=[
            pl.BlockSpec((1, 1, _BLK), lambda i: (i, 0, 0)),
            pl.BlockSpec(memory_space=pltpu.SMEM),
            pl.BlockSpec(memory_space=pltpu.SMEM),
        ],
        out_shape=[
            jax.ShapeDtypeStruct((nblk, 1, _BLK), jnp.int32),
            jax.ShapeDtypeStruct((1, 1), jnp.float32),
            jax.ShapeDtypeStruct((1, 1), jnp.float32),
        ],
        scratch_shapes=[
            pltpu.VMEM((1, _NUM_CODES), jnp.float32),
            pltpu.VMEM((1, _NUM_CODES), jnp.float32),
            pltpu.SMEM((1,), jnp.float32),
        ],
    )(slots2d, w, b2, basis)


def _sc_gather_call(table_pad, idx_flat, dim):
    nrows = idx_flat.shape[0]
    pdim = table_pad.shape[1]
    info = plsc.get_sparse_core_info()
    ncores = info.num_cores
    nw = ncores * info.num_subcores
    rpw = nrows // nw
    chunk = 16
    nch = rpw // chunk

    @functools.partial(
        pl.kernel,
        out_type=jax.ShapeDtypeStruct((nrows, pdim), jnp.float32),
        mesh=plsc.VectorSubcoreMesh(core_axis_name="c", subcore_axis_name="s"),
        scratch_types=[
            pltpu.VMEM((rpw,), jnp.int32),
            pltpu.VMEM((chunk, pdim), jnp.float32),
            pltpu.VMEM((chunk, pdim), jnp.float32),
            pltpu.SemaphoreType.DMA,
            pltpu.SemaphoreType.DMA,
        ],
    )
    def gk(table_hbm, idx_hbm, out_hbm, idx_v, buf0, buf1, s0, s1):
        wid = lax.axis_index("s") * ncores + lax.axis_index("c")
        base = wid * rpw
        pltpu.sync_copy(idx_hbm.at[pl.ds(base, rpw)], idx_v)
        bufs = (buf0, buf1)
        sems = (s0, s1)
        cps = {}
        for c in range(min(2, nch)):
            cps[c] = pltpu.async_copy(
                table_hbm.at[idx_v.at[pl.ds(c * chunk, chunk)]],
                bufs[c % 2], sems[c % 2])
        for c in range(nch):
            cps[c].wait()
            pltpu.sync_copy(bufs[c % 2],
                            out_hbm.at[pl.ds(base + c * chunk, chunk)])
            nxt = c + 2
            if nxt < nch:
                cps[nxt] = pltpu.async_copy(
                    table_hbm.at[idx_v.at[pl.ds(nxt * chunk, chunk)]],
                    bufs[nxt % 2], sems[nxt % 2])

    return gk(table_pad, idx_flat)


def kernel(slot_features, W, b, basis_vectors):
    bsz, k, d = slot_features.shape
    slots2d = slot_features.reshape(bsz * k, d)
    idx3, loss, ent = _vq_tc(slots2d, W, b.reshape(1, -1), basis_vectors)
    idx_flat = idx3.reshape(bsz * k)
    pdim = (_BASIS_DIM + 127) // 128 * 128
    table_pad = jnp.pad(basis_vectors, ((0, 0), (0, pdim - _BASIS_DIM)))
    q = _sc_gather_call(table_pad, idx_flat, _BASIS_DIM)
    q = q[:, :_BASIS_DIM]
    return (q.reshape(bsz, k, _BASIS_DIM), idx_flat.reshape(bsz, k),
            loss[0, 0], ent[0, 0])


# speedup vs baseline: 1.3344x; 1.0899x over previous
import functools

import jax
import jax.numpy as jnp
from jax import lax
from jax.experimental import pallas as pl
from jax.experimental.pallas import tpu as pltpu
from jax.experimental.pallas import tpu_sc as plsc

_NUM_CODES = 1024
_BASIS_DIM = 2700
_BETA = 0.25
_BLK = 256


_PDIM = (_BASIS_DIM + 127) // 128 * 128


def _vq_tc_body(slots_ref, w_ref, b_ref, basis_ref,
                idx_ref, loss_ref, ent_ref, tp_ref,
                acc_ref, cn_ref, msum_ref):
    i = pl.program_id(0)
    nblk = pl.num_programs(0)
    trows = _NUM_CODES // nblk

    @pl.when(i == 0)
    def _init():
        bsq = basis_ref[...] * basis_ref[...]
        cn_ref[...] = jnp.sum(bsq, axis=1)[None, :]
        acc_ref[...] = jnp.zeros_like(acc_ref)
        msum_ref[0] = 0.0

    tp_ref[0, :, :_BASIS_DIM] = basis_ref[pl.ds(i * trows, trows), :]
    tp_ref[0, :, _BASIS_DIM:] = jnp.zeros((trows, _PDIM - _BASIS_DIM),
                                          jnp.float32)

    z = jnp.dot(slots_ref[...], w_ref[...],
                preferred_element_type=jnp.float32) + b_ref[...]
    g = lax.dot_general(z, basis_ref[...], (((1,), (1,)), ((), ())),
                        preferred_element_type=jnp.float32)
    zn = jnp.sum(z * z, axis=1, keepdims=True)
    dist = zn - 2.0 * g + cn_ref[...]
    m = jnp.min(dist, axis=1, keepdims=True)
    idx_ref[0, 0, :] = jnp.argmin(dist, axis=1).astype(jnp.int32)
    p = jnp.exp(m - dist)
    p = p / jnp.sum(p, axis=1, keepdims=True)
    acc_ref[...] += jnp.sum(p, axis=0, keepdims=True)
    msum_ref[0] += jnp.sum(m)

    @pl.when(i == nblk - 1)
    def _fin():
        nrows = nblk * _BLK
        avg = acc_ref[...] / nrows
        ent_ref[0, 0] = -jnp.sum(avg * jnp.log(avg + 1e-8))
        loss_ref[0, 0] = (_BETA / (nrows * _BASIS_DIM)) * msum_ref[0]


def _vq_tc(slots2d, w, b2, basis):
    nrows, d = slots2d.shape
    nblk = nrows // _BLK
    return pl.pallas_call(
        _vq_tc_body,
        grid=(nblk,),
        in_specs=[
            pl.BlockSpec((_BLK, d), lambda i: (i, 0)),
            pl.BlockSpec(w.shape, lambda i: (0, 0)),
            pl.BlockSpec(b2.shape, lambda i: (0, 0)),
            pl.BlockSpec(basis.shape, lambda i: (0, 0)),
        ],
        out_specs=[
            pl.BlockSpec((1, 1, _BLK), lambda i: (i, 0, 0)),
            pl.BlockSpec(memory_space=pltpu.SMEM),
            pl.BlockSpec(memory_space=pltpu.SMEM),
            pl.BlockSpec((1, _NUM_CODES // nblk, _PDIM), lambda i: (i, 0, 0)),
        ],
        out_shape=[
            jax.ShapeDtypeStruct((nblk, 1, _BLK), jnp.int32),
            jax.ShapeDtypeStruct((1, 1), jnp.float32),
            jax.ShapeDtypeStruct((1, 1), jnp.float32),
            jax.ShapeDtypeStruct((nblk, _NUM_CODES // nblk, _PDIM),
                                 jnp.float32),
        ],
        scratch_shapes=[
            pltpu.VMEM((1, _NUM_CODES), jnp.float32),
            pltpu.VMEM((1, _NUM_CODES), jnp.float32),
            pltpu.SMEM((1,), jnp.float32),
        ],
    )(slots2d, w, b2, basis)


def _sc_gather_call(table_pad, idx_flat, dim):
    nrows = idx_flat.shape[0]
    pdim = table_pad.shape[1]
    info = plsc.get_sparse_core_info()
    ncores = info.num_cores
    nw = ncores * info.num_subcores
    rpw = nrows // nw
    chunk = 16
    nch = rpw // chunk

    @functools.partial(
        pl.kernel,
        out_type=jax.ShapeDtypeStruct((nrows, pdim), jnp.float32),
        mesh=plsc.VectorSubcoreMesh(core_axis_name="c", subcore_axis_name="s"),
        scratch_types=[
            pltpu.VMEM((rpw,), jnp.int32),
            pltpu.VMEM((chunk, pdim), jnp.float32),
            pltpu.VMEM((chunk, pdim), jnp.float32),
            pltpu.SemaphoreType.DMA,
            pltpu.SemaphoreType.DMA,
        ],
    )
    def gk(table_hbm, idx_hbm, out_hbm, idx_v, buf0, buf1, s0, s1):
        wid = lax.axis_index("s") * ncores + lax.axis_index("c")
        base = wid * rpw
        pltpu.sync_copy(idx_hbm.at[pl.ds(base, rpw)], idx_v)
        bufs = (buf0, buf1)
        sems = (s0, s1)
        cps = {}
        for c in range(min(2, nch)):
            cps[c] = pltpu.async_copy(
                table_hbm.at[idx_v.at[pl.ds(c * chunk, chunk)]],
                bufs[c % 2], sems[c % 2])
        for c in range(nch):
            cps[c].wait()
            pltpu.sync_copy(bufs[c % 2],
                            out_hbm.at[pl.ds(base + c * chunk, chunk)])
            nxt = c + 2
            if nxt < nch:
                cps[nxt] = pltpu.async_copy(
                    table_hbm.at[idx_v.at[pl.ds(nxt * chunk, chunk)]],
                    bufs[nxt % 2], sems[nxt % 2])

    return gk(table_pad, idx_flat)


def kernel(slot_features, W, b, basis_vectors):
    bsz, k, d = slot_features.shape
    slots2d = slot_features.reshape(bsz * k, d)
    idx3, loss, ent, tp = _vq_tc(slots2d, W, b.reshape(1, -1), basis_vectors)
    idx_flat = idx3.reshape(bsz * k)
    table_pad = tp.reshape(_NUM_CODES, _PDIM)
    q = _sc_gather_call(table_pad, idx_flat, _BASIS_DIM)
    q = q[:, :_BASIS_DIM]
    return (q.reshape(bsz, k, _BASIS_DIM), idx_flat.reshape(bsz, k),
            loss[0, 0], ent[0, 0])
